# baseline (speedup 1.0000x reference)
import jax
import jax.numpy as jnp
from jax import lax
from jax.experimental import pallas as pl
from jax.experimental.pallas import tpu as pltpu
from jax.experimental.pallas import tpu_sc as plsc

VOCAB = 1000000
SEQ = 200
EMBED = 64
BATCH = 4096
NC, NS = 2, 16
NW = NC * NS
BBLK = BATCH // NW
LANES = 16
NV = EMBED // LANES
VBLK = 4096
VGRID = -(-VOCAB // VBLK)


def _tr_body(x_ref, o_ref):
    x = x_ref[...]
    eye = jnp.eye(EMBED, dtype=jnp.float32)
    z = lax.dot_general(x, eye, (((0,), (0,)), ((), ())),
                        preferred_element_type=jnp.float32)
    z = z.reshape(VBLK // 2, 2, EMBED)
    o_ref[...] = jnp.concatenate([z[:, 0, :], z[:, 1, :]], axis=1)


_tc_transpose = pl.pallas_call(
    _tr_body,
    grid=(VGRID,),
    in_specs=[pl.BlockSpec((EMBED, VBLK), lambda i: (0, i))],
    out_specs=pl.BlockSpec((VBLK // 2, 2 * EMBED), lambda i: (i, 0)),
    out_shape=jax.ShapeDtypeStruct((VOCAB // 2, 2 * EMBED), jnp.float32),
)


def _body(tok_hbm, table_hbm, pos_hbm, out_hbm,
          tok_v, idxb_v, pos_v, rg0, rg1, ot0, ot1,
          sg0, sg1, ss0, ss1):
    w = lax.axis_index("s") * NC + lax.axis_index("c")
    rgs = (rg0, rg1)
    ots = (ot0, ot1)
    sgs = (sg0, sg1)
    sss = (ss0, ss1)

    pltpu.sync_copy(tok_hbm.at[:, pl.ds(w * BBLK, BBLK)], tok_v)
    pltpu.sync_copy(pos_hbm, pos_v)

    def cv_body(l, carry):
        for k in range(BBLK // LANES):
            sl = pl.ds(k * LANES, LANES)
            t2 = tok_v[l, sl] + tok_v[l, sl]
            tok_v[l, sl] = t2
            idxb_v[l, sl] = t2 + 1
        return carry

    lax.fori_loop(0, SEQ, cv_body, 0)

    def fire(l, pp):
        pltpu.async_copy(table_hbm.at[tok_v.at[l]], rgs[pp].at[0], sgs[pp])
        pltpu.async_copy(table_hbm.at[idxb_v.at[l]], rgs[pp].at[1], sgs[pp])

    def wait_fire(l, pp):
        pltpu.make_async_copy(table_hbm.at[tok_v.at[l]],
                              rgs[pp].at[0], sgs[pp]).wait()
        pltpu.make_async_copy(table_hbm.at[idxb_v.at[l]],
                              rgs[pp].at[1], sgs[pp]).wait()

    def fire_store(l, pp):
        pltpu.async_copy(ots[pp], out_hbm.at[l, pl.ds(w * (BBLK // 2),
                                                      BBLK // 2)], sss[pp])

    def wait_store(l, pp):
        pltpu.make_async_copy(ots[pp],
                              out_hbm.at[l, pl.ds(w * (BBLK // 2),
                                                  BBLK // 2)],
                              sss[pp]).wait()

    def compute(l, pp):
        rg = rgs[pp]
        ot = ots[pp]
        ps = tuple(pos_v[l, pl.ds(c * LANES, LANES)] for c in range(NV))

        def row_body(b2, carry):
            for half in range(2):
                for c in range(2):
                    sl = pl.ds(c * LANES, LANES)
                    v0 = rg[0, 2 * b2 + half, sl] + carry[c]
                    v1 = rg[1, 2 * b2 + half, sl] + carry[2 + c]
                    ot[b2, pl.ds(half * EMBED + c * LANES, LANES)] = v0
                    ot[b2, pl.ds(half * EMBED + 32 + c * LANES, LANES)] = v1
            return carry

        lax.fori_loop(0, BBLK // 2, row_body, ps, unroll=2)

    fire(0, 0)

    def outer(l2, carry):
        for b in range(2):
            l = 2 * l2 + b
            pp = b

            @pl.when(l >= 2)
            def _():
                wait_store(l - 2, pp)

            wait_fire(l, pp)

            @pl.when(l + 1 < SEQ)
            def _():
                fire(l + 1, 1 - pp)

            compute(l, pp)
            fire_store(l, pp)
        return carry

    lax.fori_loop(0, SEQ // 2, outer, 0)
    wait_store(SEQ - 2, 0)
    wait_store(SEQ - 1, 1)


_grid_kernel = pl.kernel(
    _body,
    out_type=jax.ShapeDtypeStruct((SEQ, BATCH // 2, 2 * EMBED), jnp.float32),
    mesh=plsc.VectorSubcoreMesh(core_axis_name="c", subcore_axis_name="s"),
    scratch_types=[
        pltpu.VMEM((SEQ, BBLK), jnp.int32),
        pltpu.VMEM((SEQ, BBLK), jnp.int32),
        pltpu.VMEM((SEQ, EMBED), jnp.float32),
        pltpu.VMEM((2, BBLK, EMBED // 2), jnp.float32),
        pltpu.VMEM((2, BBLK, EMBED // 2), jnp.float32),
        pltpu.VMEM((BBLK // 2, 2 * EMBED), jnp.float32),
        pltpu.VMEM((BBLK // 2, 2 * EMBED), jnp.float32),
        pltpu.SemaphoreType.DMA,
        pltpu.SemaphoreType.DMA,
        pltpu.SemaphoreType.DMA,
        pltpu.SemaphoreType.DMA,
    ],
    compiler_params=pltpu.CompilerParams(use_tc_tiling_on_sc=False),
)


@jax.jit
def kernel(tokens, word_table, pos_table):
    tok_t = tokens.astype(jnp.int32).T
    wtq = _tc_transpose(word_table.T)
    wt32 = wtq.reshape(2 * VOCAB, EMBED // 2)
    out3 = _grid_kernel(tok_t, wt32, pos_table)
    return out3.reshape(SEQ, BATCH, EMBED).transpose(1, 0, 2)


# speedup vs baseline: 1.0317x; 1.0317x over previous
import jax
import jax.numpy as jnp
from jax import lax
from jax.experimental import pallas as pl
from jax.experimental.pallas import tpu as pltpu
from jax.experimental.pallas import tpu_sc as plsc

VOCAB = 1000000
SEQ = 200
EMBED = 64
BATCH = 4096
NC, NS = 2, 16
NW = NC * NS
BBLK = BATCH // NW
LANES = 16
NV = EMBED // LANES
VBLK = 4096
VGRID = -(-VOCAB // VBLK)


def _tr_body(x_ref, o_ref):
    z = jnp.transpose(x_ref[...]).reshape(VBLK // 2, 2, EMBED)
    o_ref[...] = jnp.concatenate([z[:, 0, :], z[:, 1, :]], axis=1)


_tc_transpose = pl.pallas_call(
    _tr_body,
    grid=(VGRID,),
    in_specs=[pl.BlockSpec((EMBED, VBLK), lambda i: (0, i))],
    out_specs=pl.BlockSpec((VBLK // 2, 2 * EMBED), lambda i: (i, 0)),
    out_shape=jax.ShapeDtypeStruct((VOCAB // 2, 2 * EMBED), jnp.float32),
)


def _body(tok_hbm, table_hbm, pos_hbm, out_hbm,
          tok_v, idxb_v, pos_v, rg0, rg1, ot0, ot1,
          sg0, sg1, ss0, ss1):
    w = lax.axis_index("s") * NC + lax.axis_index("c")
    rgs = (rg0, rg1)
    ots = (ot0, ot1)
    sgs = (sg0, sg1)
    sss = (ss0, ss1)

    pltpu.sync_copy(tok_hbm.at[:, pl.ds(w * BBLK, BBLK)], tok_v)
    pltpu.sync_copy(pos_hbm, pos_v)

    def cv_body(l, carry):
        for k in range(BBLK // LANES):
            sl = pl.ds(k * LANES, LANES)
            t2 = tok_v[l, sl] + tok_v[l, sl]
            tok_v[l, sl] = t2
            idxb_v[l, sl] = t2 + 1
        return carry

    lax.fori_loop(0, SEQ, cv_body, 0)

    def fire(l, pp):
        pltpu.async_copy(table_hbm.at[tok_v.at[l]], rgs[pp].at[0], sgs[pp])
        pltpu.async_copy(table_hbm.at[idxb_v.at[l]], rgs[pp].at[1], sgs[pp])

    def wait_fire(l, pp):
        pltpu.make_async_copy(table_hbm.at[tok_v.at[l]],
                              rgs[pp].at[0], sgs[pp]).wait()
        pltpu.make_async_copy(table_hbm.at[idxb_v.at[l]],
                              rgs[pp].at[1], sgs[pp]).wait()

    def fire_store(l, pp):
        pltpu.async_copy(ots[pp], out_hbm.at[l, pl.ds(w * (BBLK // 2),
                                                      BBLK // 2)], sss[pp])

    def wait_store(l, pp):
        pltpu.make_async_copy(ots[pp],
                              out_hbm.at[l, pl.ds(w * (BBLK // 2),
                                                  BBLK // 2)],
                              sss[pp]).wait()

    def compute(l, pp):
        rg = rgs[pp]
        ot = ots[pp]
        ps = tuple(pos_v[l, pl.ds(c * LANES, LANES)] for c in range(NV))

        def row_body(b2, carry):
            for half in range(2):
                for c in range(2):
                    sl = pl.ds(c * LANES, LANES)
                    v0 = rg[0, 2 * b2 + half, sl] + carry[c]
                    v1 = rg[1, 2 * b2 + half, sl] + carry[2 + c]
                    ot[b2, pl.ds(half * EMBED + c * LANES, LANES)] = v0
                    ot[b2, pl.ds(half * EMBED + 32 + c * LANES, LANES)] = v1
            return carry

        lax.fori_loop(0, BBLK // 2, row_body, ps, unroll=2)

    fire(0, 0)

    def outer(l2, carry):
        for b in range(2):
            l = 2 * l2 + b
            pp = b

            @pl.when(l >= 2)
            def _():
                wait_store(l - 2, pp)

            wait_fire(l, pp)

            @pl.when(l + 1 < SEQ)
            def _():
                fire(l + 1, 1 - pp)

            compute(l, pp)
            fire_store(l, pp)
        return carry

    lax.fori_loop(0, SEQ // 2, outer, 0)
    wait_store(SEQ - 2, 0)
    wait_store(SEQ - 1, 1)


_grid_kernel = pl.kernel(
    _body,
    out_type=jax.ShapeDtypeStruct((SEQ, BATCH // 2, 2 * EMBED), jnp.float32),
    mesh=plsc.VectorSubcoreMesh(core_axis_name="c", subcore_axis_name="s"),
    scratch_types=[
        pltpu.VMEM((SEQ, BBLK), jnp.int32),
        pltpu.VMEM((SEQ, BBLK), jnp.int32),
        pltpu.VMEM((SEQ, EMBED), jnp.float32),
        pltpu.VMEM((2, BBLK, EMBED // 2), jnp.float32),
        pltpu.VMEM((2, BBLK, EMBED // 2), jnp.float32),
        pltpu.VMEM((BBLK // 2, 2 * EMBED), jnp.float32),
        pltpu.VMEM((BBLK // 2, 2 * EMBED), jnp.float32),
        pltpu.SemaphoreType.DMA,
        pltpu.SemaphoreType.DMA,
        pltpu.SemaphoreType.DMA,
        pltpu.SemaphoreType.DMA,
    ],
    compiler_params=pltpu.CompilerParams(use_tc_tiling_on_sc=False),
)


@jax.jit
def kernel(tokens, word_table, pos_table):
    tok_t = tokens.astype(jnp.int32).T
    wtq = _tc_transpose(word_table.T)
    wt32 = wtq.reshape(2 * VOCAB, EMBED // 2)
    out3 = _grid_kernel(tok_t, wt32, pos_table)
    return out3.reshape(SEQ, BATCH, EMBED).transpose(1, 0, 2)


# speedup vs baseline: 1.0555x; 1.0230x over previous
import jax
import jax.numpy as jnp
from jax import lax
from jax.experimental import pallas as pl
from jax.experimental.pallas import tpu as pltpu
from jax.experimental.pallas import tpu_sc as plsc

VOCAB = 1000000
SEQ = 200
EMBED = 64
BATCH = 4096
NC, NS = 2, 16
NW = NC * NS
BBLK = BATCH // NW
LANES = 16
NV = EMBED // LANES
VBLK = 8192
VGRID = -(-VOCAB // VBLK)


def _tr_body(x_ref, o_ref):
    z = jnp.transpose(x_ref[...]).reshape(VBLK // 2, 2, EMBED)
    o_ref[...] = jnp.concatenate([z[:, 0, :], z[:, 1, :]], axis=1)


_tc_transpose = pl.pallas_call(
    _tr_body,
    grid=(VGRID,),
    in_specs=[pl.BlockSpec((EMBED, VBLK), lambda i: (0, i))],
    out_specs=pl.BlockSpec((VBLK // 2, 2 * EMBED), lambda i: (i, 0)),
    out_shape=jax.ShapeDtypeStruct((VOCAB // 2, 2 * EMBED), jnp.float32),
)


def _body(tok_hbm, table_hbm, pos_hbm, out_hbm,
          tok_v, idxb_v, pos_v, rg0, rg1, ot0, ot1,
          sg0, sg1, ss0, ss1):
    w = lax.axis_index("s") * NC + lax.axis_index("c")
    rgs = (rg0, rg1)
    ots = (ot0, ot1)
    sgs = (sg0, sg1)
    sss = (ss0, ss1)

    pltpu.sync_copy(tok_hbm.at[:, pl.ds(w * BBLK, BBLK)], tok_v)
    pltpu.sync_copy(pos_hbm, pos_v)

    def cv_body(l, carry):
        for k in range(BBLK // LANES):
            sl = pl.ds(k * LANES, LANES)
            t2 = tok_v[l, sl] + tok_v[l, sl]
            tok_v[l, sl] = t2
            idxb_v[l, sl] = t2 + 1
        return carry

    lax.fori_loop(0, SEQ, cv_body, 0)

    def fire(l, pp):
        pltpu.async_copy(table_hbm.at[tok_v.at[l]], rgs[pp].at[0], sgs[pp])
        pltpu.async_copy(table_hbm.at[idxb_v.at[l]], rgs[pp].at[1], sgs[pp])

    def wait_fire(l, pp):
        pltpu.make_async_copy(table_hbm.at[tok_v.at[l]],
                              rgs[pp].at[0], sgs[pp]).wait()
        pltpu.make_async_copy(table_hbm.at[idxb_v.at[l]],
                              rgs[pp].at[1], sgs[pp]).wait()

    def fire_store(l, pp):
        pltpu.async_copy(ots[pp], out_hbm.at[l, pl.ds(w * (BBLK // 2),
                                                      BBLK // 2)], sss[pp])

    def wait_store(l, pp):
        pltpu.make_async_copy(ots[pp],
                              out_hbm.at[l, pl.ds(w * (BBLK // 2),
                                                  BBLK // 2)],
                              sss[pp]).wait()

    def compute(l, pp):
        rg = rgs[pp]
        ot = ots[pp]
        ps = tuple(pos_v[l, pl.ds(c * LANES, LANES)] for c in range(NV))

        def row_body(b2, carry):
            for half in range(2):
                for c in range(2):
                    sl = pl.ds(c * LANES, LANES)
                    v0 = rg[0, 2 * b2 + half, sl] + carry[c]
                    v1 = rg[1, 2 * b2 + half, sl] + carry[2 + c]
                    ot[b2, pl.ds(half * EMBED + c * LANES, LANES)] = v0
                    ot[b2, pl.ds(half * EMBED + 32 + c * LANES, LANES)] = v1
            return carry

        lax.fori_loop(0, BBLK // 2, row_body, ps, unroll=2)

    fire(0, 0)

    def outer(l2, carry):
        for b in range(2):
            l = 2 * l2 + b
            pp = b

            @pl.when(l >= 2)
            def _():
                wait_store(l - 2, pp)

            wait_fire(l, pp)

            @pl.when(l + 1 < SEQ)
            def _():
                fire(l + 1, 1 - pp)

            compute(l, pp)
            fire_store(l, pp)
        return carry

    lax.fori_loop(0, SEQ // 2, outer, 0)
    wait_store(SEQ - 2, 0)
    wait_store(SEQ - 1, 1)


_grid_kernel = pl.kernel(
    _body,
    out_type=jax.ShapeDtypeStruct((SEQ, BATCH // 2, 2 * EMBED), jnp.float32),
    mesh=plsc.VectorSubcoreMesh(core_axis_name="c", subcore_axis_name="s"),
    scratch_types=[
        pltpu.VMEM((SEQ, BBLK), jnp.int32),
        pltpu.VMEM((SEQ, BBLK), jnp.int32),
        pltpu.VMEM((SEQ, EMBED), jnp.float32),
        pltpu.VMEM((2, BBLK, EMBED // 2), jnp.float32),
        pltpu.VMEM((2, BBLK, EMBED // 2), jnp.float32),
        pltpu.VMEM((BBLK // 2, 2 * EMBED), jnp.float32),
        pltpu.VMEM((BBLK // 2, 2 * EMBED), jnp.float32),
        pltpu.SemaphoreType.DMA,
        pltpu.SemaphoreType.DMA,
        pltpu.SemaphoreType.DMA,
        pltpu.SemaphoreType.DMA,
    ],
    compiler_params=pltpu.CompilerParams(use_tc_tiling_on_sc=False),
)


@jax.jit
def kernel(tokens, word_table, pos_table):
    tok_t = tokens.astype(jnp.int32).T
    wtq = _tc_transpose(word_table.T)
    wt32 = wtq.reshape(2 * VOCAB, EMBED // 2)
    out3 = _grid_kernel(tok_t, wt32, pos_table)
    return out3.reshape(SEQ, BATCH, EMBED).transpose(1, 0, 2)


# speedup vs baseline: 1.0558x; 1.0003x over previous
import jax
import jax.numpy as jnp
from jax import lax
from jax.experimental import pallas as pl
from jax.experimental.pallas import tpu as pltpu
from jax.experimental.pallas import tpu_sc as plsc

VOCAB = 1000000
SEQ = 200
EMBED = 64
BATCH = 4096
NC, NS = 2, 16
NW = NC * NS
BBLK = BATCH // NW
LANES = 16
NV = EMBED // LANES
VBLK = 16384
VGRID = -(-VOCAB // VBLK)


def _tr_body(x_ref, o_ref):
    z = jnp.transpose(x_ref[...]).reshape(VBLK // 2, 2, EMBED)
    o_ref[...] = jnp.concatenate([z[:, 0, :], z[:, 1, :]], axis=1)


_tc_transpose = pl.pallas_call(
    _tr_body,
    grid=(VGRID,),
    in_specs=[pl.BlockSpec((EMBED, VBLK), lambda i: (0, i))],
    out_specs=pl.BlockSpec((VBLK // 2, 2 * EMBED), lambda i: (i, 0)),
    out_shape=jax.ShapeDtypeStruct((VOCAB // 2, 2 * EMBED), jnp.float32),
)


def _body(tok_hbm, table_hbm, pos_hbm, out_hbm,
          tok_v, idxb_v, pos_v, rg0, rg1, ot0, ot1,
          sg0, sg1, ss0, ss1):
    w = lax.axis_index("s") * NC + lax.axis_index("c")
    rgs = (rg0, rg1)
    ots = (ot0, ot1)
    sgs = (sg0, sg1)
    sss = (ss0, ss1)

    pltpu.sync_copy(tok_hbm.at[:, pl.ds(w * BBLK, BBLK)], tok_v)
    pltpu.sync_copy(pos_hbm, pos_v)

    def cv_body(l, carry):
        for k in range(BBLK // LANES):
            sl = pl.ds(k * LANES, LANES)
            t2 = tok_v[l, sl] + tok_v[l, sl]
            tok_v[l, sl] = t2
            idxb_v[l, sl] = t2 + 1
        return carry

    lax.fori_loop(0, SEQ, cv_body, 0)

    def fire(l, pp):
        pltpu.async_copy(table_hbm.at[tok_v.at[l]], rgs[pp].at[0], sgs[pp])
        pltpu.async_copy(table_hbm.at[idxb_v.at[l]], rgs[pp].at[1], sgs[pp])

    def wait_fire(l, pp):
        pltpu.make_async_copy(table_hbm.at[tok_v.at[l]],
                              rgs[pp].at[0], sgs[pp]).wait()
        pltpu.make_async_copy(table_hbm.at[idxb_v.at[l]],
                              rgs[pp].at[1], sgs[pp]).wait()

    def fire_store(l, pp):
        pltpu.async_copy(ots[pp], out_hbm.at[l, pl.ds(w * (BBLK // 2),
                                                      BBLK // 2)], sss[pp])

    def wait_store(l, pp):
        pltpu.make_async_copy(ots[pp],
                              out_hbm.at[l, pl.ds(w * (BBLK // 2),
                                                  BBLK // 2)],
                              sss[pp]).wait()

    def compute(l, pp):
        rg = rgs[pp]
        ot = ots[pp]
        ps = tuple(pos_v[l, pl.ds(c * LANES, LANES)] for c in range(NV))

        def row_body(b2, carry):
            for half in range(2):
                for c in range(2):
                    sl = pl.ds(c * LANES, LANES)
                    v0 = rg[0, 2 * b2 + half, sl] + carry[c]
                    v1 = rg[1, 2 * b2 + half, sl] + carry[2 + c]
                    ot[b2, pl.ds(half * EMBED + c * LANES, LANES)] = v0
                    ot[b2, pl.ds(half * EMBED + 32 + c * LANES, LANES)] = v1
            return carry

        lax.fori_loop(0, BBLK // 2, row_body, ps, unroll=2)

    fire(0, 0)

    def outer(l2, carry):
        for b in range(2):
            l = 2 * l2 + b
            pp = b

            @pl.when(l >= 2)
            def _():
                wait_store(l - 2, pp)

            wait_fire(l, pp)

            @pl.when(l + 1 < SEQ)
            def _():
                fire(l + 1, 1 - pp)

            compute(l, pp)
            fire_store(l, pp)
        return carry

    lax.fori_loop(0, SEQ // 2, outer, 0)
    wait_store(SEQ - 2, 0)
    wait_store(SEQ - 1, 1)


_grid_kernel = pl.kernel(
    _body,
    out_type=jax.ShapeDtypeStruct((SEQ, BATCH // 2, 2 * EMBED), jnp.float32),
    mesh=plsc.VectorSubcoreMesh(core_axis_name="c", subcore_axis_name="s"),
    scratch_types=[
        pltpu.VMEM((SEQ, BBLK), jnp.int32),
        pltpu.VMEM((SEQ, BBLK), jnp.int32),
        pltpu.VMEM((SEQ, EMBED), jnp.float32),
        pltpu.VMEM((2, BBLK, EMBED // 2), jnp.float32),
        pltpu.VMEM((2, BBLK, EMBED // 2), jnp.float32),
        pltpu.VMEM((BBLK // 2, 2 * EMBED), jnp.float32),
        pltpu.VMEM((BBLK // 2, 2 * EMBED), jnp.float32),
        pltpu.SemaphoreType.DMA,
        pltpu.SemaphoreType.DMA,
        pltpu.SemaphoreType.DMA,
        pltpu.SemaphoreType.DMA,
    ],
    compiler_params=pltpu.CompilerParams(use_tc_tiling_on_sc=False),
)


@jax.jit
def kernel(tokens, word_table, pos_table):
    tok_t = tokens.astype(jnp.int32).T
    wtq = _tc_transpose(word_table.T)
    wt32 = wtq.reshape(2 * VOCAB, EMBED // 2)
    out3 = _grid_kernel(tok_t, wt32, pos_table)
    return out3.reshape(SEQ, BATCH, EMBED).transpose(1, 0, 2)
